# baseline (speedup 1.0000x reference)
from math import sqrt

import jax
import jax.numpy as jnp
import numpy as np
from jax import lax
from jax.experimental import pallas as pl
from jax.experimental.pallas import tpu as pltpu
from jax.experimental.pallas import tpu_sc as plsc

D_MODEL = 768
SEQ = 2048
BATCH = 4

_INFO = plsc.get_sparse_core_info()
NC, NS, L = _INFO.num_cores, _INFO.num_subcores, _INFO.num_lanes
NW = NC * NS
S_PER_W = SEQ // NW
CH_S = 8
NJ = S_PER_W // CH_S
VPR = D_MODEL // L
SCALE = np.float32(sqrt(D_MODEL))


def _positional_encoding(max_position, d_model):
    positions = np.arange(max_position)[:, None].astype(np.float64)
    dims = np.arange(d_model)[None, :].astype(np.float64)
    angle_rates = 1.0 / np.power(10000.0, (2.0 * (dims // 2)) / float(d_model))
    angles = positions * angle_rates
    pe = np.zeros((max_position, d_model), dtype=np.float64)
    pe[:, 0::2] = np.sin(angles[:, 0::2])
    pe[:, 1::2] = np.cos(angles[:, 1::2])
    return pe.astype(np.float32)


_PE = _positional_encoding(SEQ, D_MODEL)


def _sc_body(x_hbm, emb_hbm, pe_hbm, out_hbm, idx_v, rows_v, out_v, pe_v,
             gsem, ssem, psem):
    wid = lax.axis_index("s") * NC + lax.axis_index("c")
    s0 = wid * S_PER_W
    for b in range(BATCH):
        pltpu.sync_copy(x_hbm.at[pl.ds(b * SEQ + s0, S_PER_W)], idx_v.at[b])

    def gather_copy(t, i):
        return pltpu.make_async_copy(
            emb_hbm.at[idx_v.at[i, pl.ds(t * CH_S, CH_S)]],
            rows_v.at[t % 2, i], gsem.at[t % 2])

    def pe_copy(t):
        return pltpu.make_async_copy(
            pe_hbm.at[pl.ds(s0 + t * CH_S, CH_S)], pe_v.at[t % 2],
            psem.at[t % 2])

    def store_copy(t, i):
        return pltpu.make_async_copy(
            out_v.at[t % 2, i],
            out_hbm.at[pl.ds(i * SEQ + s0 + t * CH_S, CH_S)], ssem.at[t % 2])

    for tt in range(2):
        pe_copy(tt).start()
        for i in range(BATCH):
            gather_copy(tt, i).start()

    def outer(t, carry):
        p = t % 2
        for i in range(BATCH):
            gather_copy(t, i).wait()
        pe_copy(t).wait()

        @pl.when(t >= 2)
        def _drain():
            for i in range(BATCH):
                store_copy(t - 2, i).wait()

        @plsc.parallel_loop(0, CH_S, unroll=2)
        def row_body(r):
            for c in range(VPR):
                sl = pl.ds(c * L, L)
                pvec = pe_v[p, r, sl]
                for i in range(BATCH):
                    out_v[p, i, r, sl] = rows_v[p, i, r, sl] * SCALE + pvec

        for i in range(BATCH):
            store_copy(t, i).start()

        @pl.when(t + 2 < NJ)
        def _prefetch():
            pe_copy(t + 2).start()
            for i in range(BATCH):
                gather_copy(t + 2, i).start()
        return carry

    lax.fori_loop(0, NJ, outer, 0)
    for tt in (NJ - 2, NJ - 1):
        for i in range(BATCH):
            store_copy(tt, i).wait()


@jax.jit
def _encoder(x_flat, embedding, pe):
    mesh = plsc.VectorSubcoreMesh(core_axis_name="c", subcore_axis_name="s")
    f = pl.kernel(
        _sc_body,
        out_type=jax.ShapeDtypeStruct((BATCH * SEQ, D_MODEL), jnp.float32),
        mesh=mesh,
        scratch_types=[
            pltpu.VMEM((BATCH, S_PER_W), jnp.int32),
            pltpu.VMEM((2, BATCH, CH_S, D_MODEL), jnp.float32),
            pltpu.VMEM((2, BATCH, CH_S, D_MODEL), jnp.float32),
            pltpu.VMEM((2, CH_S, D_MODEL), jnp.float32),
            pltpu.SemaphoreType.DMA((2,)),
            pltpu.SemaphoreType.DMA((2,)),
            pltpu.SemaphoreType.DMA((2,)),
        ],
    )
    return f(x_flat, embedding, pe)


def kernel(x, embedding):
    x_flat = x.reshape(BATCH * SEQ).astype(jnp.int32)
    out = _encoder(x_flat, embedding, _PE)
    return out.reshape(BATCH, SEQ, D_MODEL)


# speedup vs baseline: 2.5405x; 2.5405x over previous
from math import sqrt

import jax
import jax.numpy as jnp
import numpy as np
from jax import lax
from jax.experimental import pallas as pl
from jax.experimental.pallas import tpu as pltpu
from jax.experimental.pallas import tpu_sc as plsc

D_MODEL = 768
SEQ = 2048
BATCH = 4

_INFO = plsc.get_sparse_core_info()
NC, NS, L = _INFO.num_cores, _INFO.num_subcores, _INFO.num_lanes
NW = NC * NS
S_PER_W = SEQ // NW
CH_S = 8
NJ = S_PER_W // CH_S
VPR = D_MODEL // L
SCALE = np.float32(sqrt(D_MODEL))


def _positional_encoding(max_position, d_model):
    positions = np.arange(max_position)[:, None].astype(np.float64)
    dims = np.arange(d_model)[None, :].astype(np.float64)
    angle_rates = 1.0 / np.power(10000.0, (2.0 * (dims // 2)) / float(d_model))
    angles = positions * angle_rates
    pe = np.zeros((max_position, d_model), dtype=np.float64)
    pe[:, 0::2] = np.sin(angles[:, 0::2])
    pe[:, 1::2] = np.cos(angles[:, 1::2])
    return pe.astype(np.float32)


_PE = _positional_encoding(SEQ, D_MODEL)


def _sc_body(x_hbm, emb_hbm, pe_hbm, out_hbm, idx_v, rows_v, out_v, pe_v,
             gsem, ssem, psem):
    wid = lax.axis_index("s") * NC + lax.axis_index("c")
    s0 = wid * S_PER_W
    for b in range(BATCH):
        pltpu.sync_copy(x_hbm.at[pl.ds(b * SEQ + s0, S_PER_W)], idx_v.at[b])

    def gather_copy(t, p, i):
        return pltpu.make_async_copy(
            emb_hbm.at[idx_v.at[i, pl.ds(t * CH_S, CH_S)]],
            rows_v.at[p, i], gsem.at[p])

    def pe_copy(t, p):
        return pltpu.make_async_copy(
            pe_hbm.at[pl.ds(s0 + t * CH_S, CH_S)], pe_v.at[p], psem.at[p])

    def store_copy(t, p, i):
        return pltpu.make_async_copy(
            out_v.at[p, i],
            out_hbm.at[pl.ds(i * SEQ + s0 + t * CH_S, CH_S)], ssem.at[p])

    for up in range(2):
        pe_copy(up, up).start()
        for i in range(BATCH):
            gather_copy(up, up, i).start()

    def outer(tt, carry):
        for up in range(2):
            t = tt * 2 + up
            for i in range(BATCH):
                gather_copy(t, up, i).wait()
            pe_copy(t, up).wait()

            @pl.when(t >= 2)
            def _drain():
                for i in range(BATCH):
                    store_copy(t - 2, up, i).wait()

            @plsc.parallel_loop(0, CH_S, unroll=2)
            def row_body(r):
                for c in range(VPR):
                    sl = pl.ds(c * L, L)
                    pvec = pe_v[up, r, sl]
                    for i in range(BATCH):
                        out_v[up, i, r, sl] = rows_v[up, i, r, sl] * SCALE + pvec

            for i in range(BATCH):
                store_copy(t, up, i).start()

            @pl.when(t + 2 < NJ)
            def _prefetch():
                pe_copy(t + 2, up).start()
                for i in range(BATCH):
                    gather_copy(t + 2, up, i).start()
        return carry

    lax.fori_loop(0, NJ // 2, outer, 0)
    for up in range(2):
        for i in range(BATCH):
            store_copy(NJ - 2 + up, up, i).wait()


@jax.jit
def _encoder(x_flat, embedding, pe):
    mesh = plsc.VectorSubcoreMesh(core_axis_name="c", subcore_axis_name="s")
    f = pl.kernel(
        _sc_body,
        out_type=jax.ShapeDtypeStruct((BATCH * SEQ, D_MODEL), jnp.float32),
        mesh=mesh,
        scratch_types=[
            pltpu.VMEM((BATCH, S_PER_W), jnp.int32),
            pltpu.VMEM((2, BATCH, CH_S, D_MODEL), jnp.float32),
            pltpu.VMEM((2, BATCH, CH_S, D_MODEL), jnp.float32),
            pltpu.VMEM((2, CH_S, D_MODEL), jnp.float32),
            pltpu.SemaphoreType.DMA((2,)),
            pltpu.SemaphoreType.DMA((2,)),
            pltpu.SemaphoreType.DMA((2,)),
        ],
    )
    return f(x_flat, embedding, pe)


def kernel(x, embedding):
    x_flat = x.reshape(BATCH * SEQ).astype(jnp.int32)
    out = _encoder(x_flat, embedding, _PE)
    return out.reshape(BATCH, SEQ, D_MODEL)
